# baseline (speedup 1.0000x reference)
import functools

import jax
import jax.numpy as jnp
from jax import lax
from jax.experimental import pallas as pl
from jax.experimental.pallas import tpu as pltpu
from jax.experimental.pallas import tpu_sc as plsc

N_NODES_P = 10000
N_EDGES_P = 320000
D_NODE_P = 128
D_EDGE_P = 16
D_OUT_P = D_NODE_P + D_EDGE_P


def _make_proj_t_body(x_contract_dim):
    def _proj_t_body(w_ref, x_ref, b_ref, o_ref):
        o_ref[...] = (
            lax.dot_general(
                w_ref[...], x_ref[...], (((0,), (x_contract_dim,)), ((), ())),
                preferred_element_type=jnp.float32,
            )
            + b_ref[...]
        )

    return _proj_t_body


def _project_t(x, W, b, block_cols, x_transposed):
    if x_transposed:
        d_in, n = x.shape
        x_dim = 0
        x_spec = pl.BlockSpec((d_in, block_cols), lambda i: (0, i))
    else:
        n, d_in = x.shape
        x_dim = 1
        x_spec = pl.BlockSpec((block_cols, d_in), lambda i: (i, 0))
    d_out = W.shape[1]
    grid = n // block_cols
    return pl.pallas_call(
        _make_proj_t_body(x_dim),
        grid=(grid,),
        in_specs=[
            pl.BlockSpec((d_in, d_out), lambda i: (0, 0)),
            x_spec,
            pl.BlockSpec((d_out, 1), lambda i: (0, 0)),
        ],
        out_specs=pl.BlockSpec((d_out, block_cols), lambda i: (0, i)),
        out_shape=jax.ShapeDtypeStruct((d_out, n), jnp.float32),
    )(W, x, b.reshape(d_out, 1))


_NC = 2
_NS = 16
_NW = _NC * _NS
_RG = 8
_NG = D_NODE_P // _RG
_NH = _NW // _NG
_EH = N_EDGES_P // _NH
_NP = 10240
_NT = _NP // 128
_C = 1280
_CT = _C // 128
_NCH = _EH // _C
_CP = 1280
_NPT = 25
_EPT = N_EDGES_P // _NPT
_NPC = _EPT // _CP


def _sc_assemble(hv_t, idx, hp_t):
    mesh = plsc.VectorSubcoreMesh(core_axis_name="c", subcore_axis_name="s")

    @functools.partial(
        pl.kernel,
        out_type=jax.ShapeDtypeStruct((D_OUT_P, N_EDGES_P), jnp.float32),
        mesh=mesh,
        compiler_params=pltpu.CompilerParams(needs_layout_passes=False),
        scratch_types=[
            pltpu.VMEM((_NT, _RG, 128), jnp.float32),
            pltpu.VMEM((_C,), jnp.int32),
            pltpu.VMEM((_CT, _RG, 128), jnp.float32),
            pltpu.VMEM((2, _CP // 128, 8, 128), jnp.float32),
        ],
    )
    def body(hv_hbm, idx_hbm, hp_hbm, out_hbm, table_v, idx_v, outb_v, pbuf_v):
        wid = lax.axis_index("s") * _NC + lax.axis_index("c")
        g = wid % _NG
        h = wid // _NG

        def stage(c, carry):
            pltpu.sync_copy(
                hv_hbm.at[pl.ds(g * _RG, _RG), pl.ds(c * 128, 128)],
                table_v.at[c],
            )
            return carry

        lax.fori_loop(0, _NT, stage, 0)

        def gather_chunk(i, carry):
            off = h * _EH + i * _C
            pltpu.sync_copy(idx_hbm.at[pl.ds(off, _C)], idx_v)

            def grp(j, carry2):
                iv = idx_v[pl.ds(j * 16, 16)]
                ct = lax.shift_right_logical(iv, 7)
                lane = lax.bitwise_and(iv, 127)
                ot = j // 8
                ol = (j % 8) * 16
                for f in range(_RG):
                    fv = jnp.full((16,), f, jnp.int32)
                    outb_v[ot, f, pl.ds(ol, 16)] = plsc.load_gather(
                        table_v, [ct, fv, lane]
                    )
                return carry2

            lax.fori_loop(0, _C // 16, grp, 0)
            for t in range(_CT):
                pltpu.sync_copy(
                    outb_v.at[t],
                    out_hbm.at[pl.ds(g * _RG, _RG), pl.ds(off + t * 128, 128)],
                )
            return carry

        lax.fori_loop(0, _NCH, gather_chunk, 0)

        def proj_chunk(i, carry):
            off = wid * _EPT + i * _CP
            for k in range(2):
                for t in range(_CP // 128):
                    pltpu.sync_copy(
                        hp_hbm.at[pl.ds(k * 8, 8), pl.ds(off + t * 128, 128)],
                        pbuf_v.at[k, t],
                    )
            for k in range(2):
                for t in range(_CP // 128):
                    pltpu.sync_copy(
                        pbuf_v.at[k, t],
                        out_hbm.at[
                            pl.ds(D_NODE_P + k * 8, 8),
                            pl.ds(off + t * 128, 128),
                        ],
                    )
            return carry

        @pl.when(wid < _NPT)
        def _():
            lax.fori_loop(0, _NPC, proj_chunk, 0)

    return body(hv_t, idx, hp_t)


def kernel(node_feats, edge_index, edge_feats, W_n, b_n, W_e, b_e):
    src = edge_index[0].astype(jnp.int32)
    x_pad = jnp.pad(node_feats, ((0, _NP - N_NODES_P), (0, 0)))
    hv_t = _project_t(x_pad, W_n, b_n, _NP, x_transposed=False)
    hp_t = _project_t(edge_feats.T, W_e, b_e, 32000, x_transposed=True)
    out_t = _sc_assemble(hv_t, src, hp_t)
    return out_t.T


# speedup vs baseline: 1.7175x; 1.7175x over previous
import functools

import jax
import jax.numpy as jnp
from jax import lax
from jax.experimental import pallas as pl
from jax.experimental.pallas import tpu as pltpu
from jax.experimental.pallas import tpu_sc as plsc

N_NODES_P = 10000
N_EDGES_P = 320000
D_NODE_P = 128
D_EDGE_P = 16
D_OUT_P = D_NODE_P + D_EDGE_P


def _make_proj_t_body(x_contract_dim):
    def _proj_t_body(w_ref, x_ref, b_ref, o_ref):
        o_ref[...] = (
            lax.dot_general(
                w_ref[...], x_ref[...], (((0,), (x_contract_dim,)), ((), ())),
                preferred_element_type=jnp.float32,
            )
            + b_ref[...]
        )

    return _proj_t_body


def _project_t(x, W, b, block_cols, x_transposed):
    if x_transposed:
        d_in, n = x.shape
        x_dim = 0
        x_spec = pl.BlockSpec((d_in, block_cols), lambda i: (0, i))
    else:
        n, d_in = x.shape
        x_dim = 1
        x_spec = pl.BlockSpec((block_cols, d_in), lambda i: (i, 0))
    d_out = W.shape[1]
    grid = n // block_cols
    return pl.pallas_call(
        _make_proj_t_body(x_dim),
        grid=(grid,),
        in_specs=[
            pl.BlockSpec((d_in, d_out), lambda i: (0, 0)),
            x_spec,
            pl.BlockSpec((d_out, 1), lambda i: (0, 0)),
        ],
        out_specs=pl.BlockSpec((d_out, block_cols), lambda i: (0, i)),
        out_shape=jax.ShapeDtypeStruct((d_out, n), jnp.float32),
    )(W, x, b.reshape(d_out, 1))


_NC = 2
_NS = 16
_NW = _NC * _NS
_RG = 8
_NG = D_NODE_P // _RG
_NH = _NW // _NG
_EH = N_EDGES_P // _NH
_NP = 10240
_NT = _NP // 128
_C = 1280
_CT = _C // 128
_NCH = _EH // _C
_CP = 1280
_NPT = 25
_EPT = N_EDGES_P // _NPT
_NPC = _EPT // _CP


def _sc_assemble(hv_t, idx, hp_t):
    mesh = plsc.VectorSubcoreMesh(core_axis_name="c", subcore_axis_name="s")

    @functools.partial(
        pl.kernel,
        out_type=jax.ShapeDtypeStruct((D_OUT_P, N_EDGES_P), jnp.float32),
        mesh=mesh,
        compiler_params=pltpu.CompilerParams(needs_layout_passes=False),
        scratch_types=[
            pltpu.VMEM((_RG * _NP,), jnp.float32),
            pltpu.VMEM((_C,), jnp.int32),
            pltpu.VMEM((_CT, _RG, 128), jnp.float32),
            pltpu.VMEM((2, _CP // 128, 8, 128), jnp.float32),
        ],
    )
    def body(hv_hbm, idx_hbm, hp_hbm, out_hbm, table_v, idx_v, outb_v, pbuf_v):
        wid = lax.axis_index("s") * _NC + lax.axis_index("c")
        g = wid % _NG
        h = wid // _NG

        for f in range(_RG):
            pltpu.sync_copy(
                hv_hbm.at[pl.ds((g * _RG + f) * _NP, _NP)],
                table_v.at[pl.ds(f * _NP, _NP)],
            )

        def gather_chunk(i, carry):
            off = h * _EH + i * _C
            pltpu.sync_copy(idx_hbm.at[pl.ds(off, _C)], idx_v)

            def grp(j):
                iv = idx_v[pl.ds(j * 16, 16)]
                ot = j // 8
                ol = (j % 8) * 16
                for f in range(_RG):
                    outb_v[ot, f, pl.ds(ol, 16)] = plsc.load_gather(
                        table_v, [iv + (f * _NP)]
                    )

            plsc.parallel_loop(0, _C // 16, 1, unroll=8)(grp)
            for t in range(_CT):
                pltpu.sync_copy(
                    outb_v.at[t],
                    out_hbm.at[pl.ds(g * _RG, _RG), pl.ds(off + t * 128, 128)],
                )
            return carry

        lax.fori_loop(0, _NCH, gather_chunk, 0)

        def proj_chunk(i, carry):
            off = wid * _EPT + i * _CP
            for k in range(2):
                for t in range(_CP // 128):
                    pltpu.sync_copy(
                        hp_hbm.at[pl.ds(k * 8, 8), pl.ds(off + t * 128, 128)],
                        pbuf_v.at[k, t],
                    )
            for k in range(2):
                for t in range(_CP // 128):
                    pltpu.sync_copy(
                        pbuf_v.at[k, t],
                        out_hbm.at[
                            pl.ds(D_NODE_P + k * 8, 8),
                            pl.ds(off + t * 128, 128),
                        ],
                    )
            return carry

        @pl.when(wid < _NPT)
        def _():
            lax.fori_loop(0, _NPC, proj_chunk, 0)

    return body(hv_t, idx, hp_t)


def kernel(node_feats, edge_index, edge_feats, W_n, b_n, W_e, b_e):
    src = edge_index[0].astype(jnp.int32)
    x_pad = jnp.pad(node_feats, ((0, _NP - N_NODES_P), (0, 0)))
    hv_t = _project_t(x_pad, W_n, b_n, _NP, x_transposed=False)
    hp_t = _project_t(edge_feats.T, W_e, b_e, 32000, x_transposed=True)
    out_t = _sc_assemble(hv_t.reshape(-1), src, hp_t)
    return out_t.T


# speedup vs baseline: 2.5884x; 1.5071x over previous
import functools

import jax
import jax.numpy as jnp
from jax import lax
from jax.experimental import pallas as pl
from jax.experimental.pallas import tpu as pltpu
from jax.experimental.pallas import tpu_sc as plsc

N_NODES_P = 10000
N_EDGES_P = 320000
D_NODE_P = 128
D_EDGE_P = 16
D_OUT_P = D_NODE_P + D_EDGE_P


def _make_proj_t_body(x_contract_dim):
    def _proj_t_body(w_ref, x_ref, b_ref, o_ref):
        o_ref[...] = (
            lax.dot_general(
                w_ref[...], x_ref[...], (((0,), (x_contract_dim,)), ((), ())),
                preferred_element_type=jnp.float32,
            )
            + b_ref[...]
        )

    return _proj_t_body


def _project_t(x, W, b, block_cols, x_transposed):
    if x_transposed:
        d_in, n = x.shape
        x_dim = 0
        x_spec = pl.BlockSpec((d_in, block_cols), lambda i: (0, i))
    else:
        n, d_in = x.shape
        x_dim = 1
        x_spec = pl.BlockSpec((block_cols, d_in), lambda i: (i, 0))
    d_out = W.shape[1]
    grid = n // block_cols
    return pl.pallas_call(
        _make_proj_t_body(x_dim),
        grid=(grid,),
        in_specs=[
            pl.BlockSpec((d_in, d_out), lambda i: (0, 0)),
            x_spec,
            pl.BlockSpec((d_out, 1), lambda i: (0, 0)),
        ],
        out_specs=pl.BlockSpec((d_out, block_cols), lambda i: (0, i)),
        out_shape=jax.ShapeDtypeStruct((d_out, n), jnp.float32),
    )(W, x, b.reshape(d_out, 1))


_NC = 2
_NS = 16
_NW = _NC * _NS
_RG = 8
_NG = D_NODE_P // _RG
_NH = _NW // _NG
_EH = N_EDGES_P // _NH
_NP = 10240
_NT = _NP // 128
_C = 1280
_CT = _C // 128
_NCH = _EH // _C
_CP = 1280
_NPT = 25
_EPT = N_EDGES_P // _NPT
_NPC = _EPT // _CP


def _sc_assemble(hv_t, idx, hp_t):
    mesh = plsc.VectorSubcoreMesh(core_axis_name="c", subcore_axis_name="s")

    @functools.partial(
        pl.kernel,
        out_type=jax.ShapeDtypeStruct((D_OUT_P, N_EDGES_P), jnp.float32),
        mesh=mesh,
        compiler_params=pltpu.CompilerParams(needs_layout_passes=False),
        scratch_types=[
            pltpu.VMEM((_RG * _NP,), jnp.float32),
            pltpu.VMEM((_C,), jnp.int32),
            pltpu.VMEM((_RG, _C), jnp.float32),
            pltpu.VMEM((8, _CP), jnp.float32),
            pltpu.VMEM((8, _CP), jnp.float32),
        ],
    )
    def body(
        hv_hbm, idx_hbm, hp_hbm, out_hbm, table_v, idx_v, outb_v, pb0_v, pb1_v
    ):
        wid = lax.axis_index("s") * _NC + lax.axis_index("c")
        g = wid % _NG
        h = wid // _NG

        for f in range(_RG):
            pltpu.sync_copy(
                hv_hbm.at[pl.ds((g * _RG + f) * _NP, _NP)],
                table_v.at[pl.ds(f * _NP, _NP)],
            )

        def gather_chunk(i, carry):
            off = h * _EH + i * _C
            pltpu.sync_copy(idx_hbm.at[pl.ds(off, _C)], idx_v)

            def grp(j):
                iv = idx_v[pl.ds(j * 16, 16)]
                for f in range(_RG):
                    outb_v[f, pl.ds(j * 16, 16)] = plsc.load_gather(
                        table_v, [iv + (f * _NP)]
                    )

            plsc.parallel_loop(0, _C // 16, 1, unroll=8)(grp)
            pltpu.sync_copy(
                outb_v, out_hbm.at[pl.ds(g * _RG, _RG), pl.ds(off, _C)]
            )
            return carry

        lax.fori_loop(0, _NCH, gather_chunk, 0)

        def proj_chunk(i, carry):
            off = wid * _EPT + i * _CP
            pltpu.sync_copy(hp_hbm.at[pl.ds(0, 8), pl.ds(off, _CP)], pb0_v)
            pltpu.sync_copy(hp_hbm.at[pl.ds(8, 8), pl.ds(off, _CP)], pb1_v)
            pltpu.sync_copy(
                pb0_v, out_hbm.at[pl.ds(D_NODE_P, 8), pl.ds(off, _CP)]
            )
            pltpu.sync_copy(
                pb1_v, out_hbm.at[pl.ds(D_NODE_P + 8, 8), pl.ds(off, _CP)]
            )
            return carry

        @pl.when(wid < _NPT)
        def _():
            lax.fori_loop(0, _NPC, proj_chunk, 0)

    return body(hv_t, idx, hp_t)


def kernel(node_feats, edge_index, edge_feats, W_n, b_n, W_e, b_e):
    src = edge_index[0].astype(jnp.int32)
    x_pad = jnp.pad(node_feats, ((0, _NP - N_NODES_P), (0, 0)))
    hv_t = _project_t(x_pad, W_n, b_n, _NP, x_transposed=False)
    hp_t = _project_t(edge_feats.T, W_e, b_e, 32000, x_transposed=True)
    out_t = _sc_assemble(hv_t.reshape(-1), src, hp_t)
    return out_t.T


# speedup vs baseline: 4.1030x; 1.5852x over previous
import functools

import jax
import jax.numpy as jnp
from jax import lax
from jax.experimental import pallas as pl
from jax.experimental.pallas import tpu as pltpu
from jax.experimental.pallas import tpu_sc as plsc

N_NODES_P = 10000
N_EDGES_P = 320000
D_NODE_P = 128
D_EDGE_P = 16
D_OUT_P = D_NODE_P + D_EDGE_P


def _make_proj_t_body(x_contract_dim):
    def _proj_t_body(w_ref, x_ref, b_ref, o_ref):
        o_ref[...] = (
            lax.dot_general(
                w_ref[...], x_ref[...], (((0,), (x_contract_dim,)), ((), ())),
                preferred_element_type=jnp.float32,
            )
            + b_ref[...]
        )

    return _proj_t_body


def _project_t(x, W, b, block_cols, x_transposed):
    if x_transposed:
        d_in, n = x.shape
        x_dim = 0
        x_spec = pl.BlockSpec((d_in, block_cols), lambda i: (0, i))
    else:
        n, d_in = x.shape
        x_dim = 1
        x_spec = pl.BlockSpec((block_cols, d_in), lambda i: (i, 0))
    d_out = W.shape[1]
    grid = n // block_cols
    return pl.pallas_call(
        _make_proj_t_body(x_dim),
        grid=(grid,),
        in_specs=[
            pl.BlockSpec((d_in, d_out), lambda i: (0, 0)),
            x_spec,
            pl.BlockSpec((d_out, 1), lambda i: (0, 0)),
        ],
        out_specs=pl.BlockSpec((d_out, block_cols), lambda i: (0, i)),
        out_shape=jax.ShapeDtypeStruct((d_out, n), jnp.float32),
    )(W, x, b.reshape(d_out, 1))


_NC = 2
_NS = 16
_NW = _NC * _NS
_RG = 8
_NG = D_NODE_P // _RG
_NH = _NW // _NG
_EH = N_EDGES_P // _NH
_NP = 10240
_NT = _NP // 128
_C = 1280
_CT = _C // 128
_NCH = _EH // _C
_CP = 1280
_NPT = 25
_EPT = N_EDGES_P // _NPT
_NPC = _EPT // _CP


def _sc_assemble(hv_t, idx, hp_t):
    mesh = plsc.VectorSubcoreMesh(core_axis_name="c", subcore_axis_name="s")

    @functools.partial(
        pl.kernel,
        out_type=jax.ShapeDtypeStruct((D_OUT_P, N_EDGES_P), jnp.float32),
        mesh=mesh,
        compiler_params=pltpu.CompilerParams(needs_layout_passes=False),
        scratch_types=[
            pltpu.VMEM((_RG * _NP,), jnp.float32),
            pltpu.VMEM((_C,), jnp.int32),
            pltpu.VMEM((_C,), jnp.int32),
            pltpu.VMEM((_RG, _C), jnp.float32),
            pltpu.VMEM((_RG, _C), jnp.float32),
            pltpu.VMEM((8, _CP), jnp.float32),
            pltpu.VMEM((8, _CP), jnp.float32),
            pltpu.SemaphoreType.DMA,
            pltpu.SemaphoreType.DMA,
            pltpu.SemaphoreType.DMA,
            pltpu.SemaphoreType.DMA,
            pltpu.SemaphoreType.DMA,
            pltpu.SemaphoreType.DMA,
        ],
    )
    def body(
        hv_hbm, idx_hbm, hp_hbm, out_hbm,
        table_v, ix0, ix1, ob0, ob1, pb0, pb1,
        sI0, sI1, sO0, sO1, sPi, sPo,
    ):
        wid = lax.axis_index("s") * _NC + lax.axis_index("c")
        g = wid % _NG
        h = wid // _NG

        for f in range(_RG):
            pltpu.sync_copy(
                hv_hbm.at[pl.ds((g * _RG + f) * _NP, _NP)],
                table_v.at[pl.ds(f * _NP, _NP)],
            )

        def idx_start(c, buf, sem):
            pltpu.async_copy(idx_hbm.at[pl.ds(h * _EH + c * _C, _C)], buf, sem)

        def idx_wait(buf, sem):
            pltpu.make_async_copy(idx_hbm.at[pl.ds(0, _C)], buf, sem).wait()

        def out_start(c, buf, sem):
            pltpu.async_copy(
                buf,
                out_hbm.at[pl.ds(g * _RG, _RG), pl.ds(h * _EH + c * _C, _C)],
                sem,
            )

        def out_wait(buf, sem):
            pltpu.make_async_copy(
                out_hbm.at[pl.ds(0, _RG), pl.ds(0, _C)], buf, sem
            ).wait()

        def gather_into(buf_i, buf_o):
            def grp(j):
                iv = buf_i[pl.ds(j * 16, 16)]
                for f in range(_RG):
                    buf_o[f, pl.ds(j * 16, 16)] = plsc.load_gather(
                        table_v, [iv + (f * _NP)]
                    )

            plsc.parallel_loop(0, _C // 16, 1, unroll=8)(grp)

        idx_start(0, ix0, sI0)

        def pair(i2, carry):
            a = 2 * i2
            idx_start(a + 1, ix1, sI1)
            idx_wait(ix0, sI0)

            @pl.when(i2 > 0)
            def _():
                out_wait(ob0, sO0)

            gather_into(ix0, ob0)
            out_start(a, ob0, sO0)
            idx_start(a + 2, ix0, sI0)
            idx_wait(ix1, sI1)

            @pl.when(i2 > 0)
            def _():
                out_wait(ob1, sO1)

            gather_into(ix1, ob1)
            out_start(a + 1, ob1, sO1)
            return carry

        lax.fori_loop(0, (_NCH - 1) // 2, pair, 0)

        idx_wait(ix0, sI0)
        out_wait(ob0, sO0)
        gather_into(ix0, ob0)
        out_start(_NCH - 1, ob0, sO0)
        out_wait(ob0, sO0)
        out_wait(ob1, sO1)

        def pj_wait(buf, sem):
            pltpu.make_async_copy(
                hp_hbm.at[pl.ds(0, 8), pl.ds(0, _CP)], buf, sem
            ).wait()

        def proj_chunk(i, carry):
            off = wid * _EPT + i * _CP

            @pl.when(i > 0)
            def _():
                pj_wait(pb0, sPo)
                pj_wait(pb1, sPo)

            pltpu.async_copy(hp_hbm.at[pl.ds(0, 8), pl.ds(off, _CP)], pb0, sPi)
            pltpu.async_copy(hp_hbm.at[pl.ds(8, 8), pl.ds(off, _CP)], pb1, sPi)
            pj_wait(pb0, sPi)
            pj_wait(pb1, sPi)
            pltpu.async_copy(
                pb0, out_hbm.at[pl.ds(D_NODE_P, 8), pl.ds(off, _CP)], sPo
            )
            pltpu.async_copy(
                pb1, out_hbm.at[pl.ds(D_NODE_P + 8, 8), pl.ds(off, _CP)], sPo
            )
            return carry

        @pl.when(wid < _NPT)
        def _():
            lax.fori_loop(0, _NPC, proj_chunk, 0)
            pj_wait(pb0, sPo)
            pj_wait(pb1, sPo)

    return body(hv_t, idx, hp_t)


def kernel(node_feats, edge_index, edge_feats, W_n, b_n, W_e, b_e):
    src = edge_index[0].astype(jnp.int32)
    x_pad = jnp.pad(node_feats, ((0, _NP - N_NODES_P), (0, 0)))
    hv_t = _project_t(x_pad, W_n, b_n, _NP, x_transposed=False)
    hp_t = _project_t(edge_feats.T, W_e, b_e, 32000, x_transposed=True)
    out_t = _sc_assemble(hv_t.reshape(-1), src, hp_t)
    return out_t.T


# speedup vs baseline: 4.3904x; 1.0700x over previous
import functools

import jax
import jax.numpy as jnp
from jax import lax
from jax.experimental import pallas as pl
from jax.experimental.pallas import tpu as pltpu
from jax.experimental.pallas import tpu_sc as plsc

N_NODES_P = 10000
N_EDGES_P = 320000
D_NODE_P = 128
D_EDGE_P = 16
D_OUT_P = D_NODE_P + D_EDGE_P


def _make_proj_t_body(x_contract_dim):
    def _proj_t_body(w_ref, x_ref, b_ref, o_ref):
        o_ref[...] = (
            lax.dot_general(
                w_ref[...], x_ref[...], (((0,), (x_contract_dim,)), ((), ())),
                preferred_element_type=jnp.float32,
            )
            + b_ref[...]
        )

    return _proj_t_body


def _project_t(x, W, b, block_cols, x_transposed):
    if x_transposed:
        d_in, n = x.shape
        x_dim = 0
        x_spec = pl.BlockSpec((d_in, block_cols), lambda i: (0, i))
    else:
        n, d_in = x.shape
        x_dim = 1
        x_spec = pl.BlockSpec((block_cols, d_in), lambda i: (i, 0))
    d_out = W.shape[1]
    grid = n // block_cols
    return pl.pallas_call(
        _make_proj_t_body(x_dim),
        grid=(grid,),
        in_specs=[
            pl.BlockSpec((d_in, d_out), lambda i: (0, 0)),
            x_spec,
            pl.BlockSpec((d_out, 1), lambda i: (0, 0)),
        ],
        out_specs=pl.BlockSpec((d_out, block_cols), lambda i: (0, i)),
        out_shape=jax.ShapeDtypeStruct((d_out, n), jnp.float32),
    )(W, x, b.reshape(d_out, 1))


_NC = 2
_NS = 16
_NW = _NC * _NS
_RG = 8
_NG = D_NODE_P // _RG
_NH = _NW // _NG
_EH = N_EDGES_P // _NH
_NP = 10240
_NT = _NP // 128
_C = 1280
_CT = _C // 128
_NCH = _EH // _C
_CP = 640
_NPT = 25
_EPT = N_EDGES_P // _NPT
_NPI = _EPT // (2 * _CP)


def _sc_assemble(hv_t, idx, hp_t):
    mesh = plsc.VectorSubcoreMesh(core_axis_name="c", subcore_axis_name="s")

    @functools.partial(
        pl.kernel,
        out_type=jax.ShapeDtypeStruct((D_OUT_P, N_EDGES_P), jnp.float32),
        mesh=mesh,
        compiler_params=pltpu.CompilerParams(needs_layout_passes=False),
        scratch_types=[
            pltpu.VMEM((_RG * _NP,), jnp.float32),
            pltpu.VMEM((_C,), jnp.int32),
            pltpu.VMEM((_C,), jnp.int32),
            pltpu.VMEM((_RG, _C), jnp.float32),
            pltpu.VMEM((_RG, _C), jnp.float32),
            pltpu.VMEM((8, _CP), jnp.float32),
            pltpu.VMEM((8, _CP), jnp.float32),
            pltpu.VMEM((8, _CP), jnp.float32),
            pltpu.VMEM((8, _CP), jnp.float32),
            pltpu.SemaphoreType.DMA,
            pltpu.SemaphoreType.DMA,
            pltpu.SemaphoreType.DMA,
            pltpu.SemaphoreType.DMA,
            pltpu.SemaphoreType.DMA,
            pltpu.SemaphoreType.DMA,
        ],
    )
    def body(
        hv_hbm, idx_hbm, hp_hbm, out_hbm,
        table_v, ix0, ix1, ob0, ob1, pa0, pa1, pb0, pb1,
        sI0, sI1, sO0, sO1, sPi, sPo,
    ):
        wid = lax.axis_index("s") * _NC + lax.axis_index("c")
        g = wid % _NG
        h = wid // _NG

        for f in range(_RG):
            pltpu.async_copy(
                hv_hbm.at[pl.ds((g * _RG + f) * _NP, _NP)],
                table_v.at[pl.ds(f * _NP, _NP)],
                sO0,
            )

        def idx_start(c, buf, sem):
            pltpu.async_copy(idx_hbm.at[pl.ds(h * _EH + c * _C, _C)], buf, sem)

        def idx_wait(buf, sem):
            pltpu.make_async_copy(idx_hbm.at[pl.ds(0, _C)], buf, sem).wait()

        def out_start(c, buf, sem):
            pltpu.async_copy(
                buf,
                out_hbm.at[pl.ds(g * _RG, _RG), pl.ds(h * _EH + c * _C, _C)],
                sem,
            )

        def out_wait(buf, sem):
            pltpu.make_async_copy(
                out_hbm.at[pl.ds(0, _RG), pl.ds(0, _C)], buf, sem
            ).wait()

        def gather_into(buf_i, buf_o):
            def grp(j):
                iv = buf_i[pl.ds(j * 16, 16)]
                for f in range(_RG):
                    buf_o[f, pl.ds(j * 16, 16)] = plsc.load_gather(
                        table_v, [iv + (f * _NP)]
                    )

            plsc.parallel_loop(0, _C // 16, 1, unroll=8)(grp)

        is_proj = wid < _NPT

        def pj_wait(buf, sem):
            pltpu.make_async_copy(
                hp_hbm.at[pl.ds(0, 8), pl.ds(0, _CP)], buf, sem
            ).wait()

        def pj_wait4(sem):
            pj_wait(pa0, sem)
            pj_wait(pa1, sem)
            pj_wait(pb0, sem)
            pj_wait(pb1, sem)

        def pj_in(off):
            pltpu.async_copy(hp_hbm.at[pl.ds(0, 8), pl.ds(off, _CP)], pa0, sPi)
            pltpu.async_copy(hp_hbm.at[pl.ds(8, 8), pl.ds(off, _CP)], pa1, sPi)
            off2 = off + _CP
            pltpu.async_copy(hp_hbm.at[pl.ds(0, 8), pl.ds(off2, _CP)], pb0, sPi)
            pltpu.async_copy(hp_hbm.at[pl.ds(8, 8), pl.ds(off2, _CP)], pb1, sPi)

        def pj_out(off):
            pltpu.async_copy(
                pa0, out_hbm.at[pl.ds(D_NODE_P, 8), pl.ds(off, _CP)], sPo
            )
            pltpu.async_copy(
                pa1, out_hbm.at[pl.ds(D_NODE_P + 8, 8), pl.ds(off, _CP)], sPo
            )
            off2 = off + _CP
            pltpu.async_copy(
                pb0, out_hbm.at[pl.ds(D_NODE_P, 8), pl.ds(off2, _CP)], sPo
            )
            pltpu.async_copy(
                pb1, out_hbm.at[pl.ds(D_NODE_P + 8, 8), pl.ds(off2, _CP)], sPo
            )

        idx_start(0, ix0, sI0)
        for f in range(_RG):
            pltpu.make_async_copy(
                hv_hbm.at[pl.ds(0, _NP)],
                table_v.at[pl.ds(f * _NP, _NP)],
                sO0,
            ).wait()

        def pair(i2, carry):
            a = 2 * i2
            idx_start(a + 1, ix1, sI1)

            @pl.when(is_proj & (i2 > 0) & (i2 <= _NPI))
            def _():
                pj_wait4(sPo)

            @pl.when(is_proj & (i2 < _NPI))
            def _():
                pj_in(wid * _EPT + i2 * 2 * _CP)

            idx_wait(ix0, sI0)

            @pl.when(i2 > 0)
            def _():
                out_wait(ob0, sO0)

            gather_into(ix0, ob0)
            out_start(a, ob0, sO0)
            idx_start(a + 2, ix0, sI0)

            @pl.when(is_proj & (i2 < _NPI))
            def _():
                pj_wait4(sPi)
                pj_out(wid * _EPT + i2 * 2 * _CP)

            idx_wait(ix1, sI1)

            @pl.when(i2 > 0)
            def _():
                out_wait(ob1, sO1)

            gather_into(ix1, ob1)
            out_start(a + 1, ob1, sO1)
            return carry

        lax.fori_loop(0, (_NCH - 1) // 2, pair, 0)

        idx_wait(ix0, sI0)
        out_wait(ob0, sO0)
        gather_into(ix0, ob0)
        out_start(_NCH - 1, ob0, sO0)
        out_wait(ob0, sO0)
        out_wait(ob1, sO1)

    return body(hv_t, idx, hp_t)


def kernel(node_feats, edge_index, edge_feats, W_n, b_n, W_e, b_e):
    src = edge_index[0].astype(jnp.int32)
    x_pad = jnp.pad(node_feats, ((0, _NP - N_NODES_P), (0, 0)))
    hv_t = _project_t(x_pad, W_n, b_n, _NP, x_transposed=False)
    hp_t = _project_t(edge_feats.T, W_e, b_e, 32000, x_transposed=True)
    out_t = _sc_assemble(hv_t.reshape(-1), src, hp_t)
    return out_t.T
